# baseline (speedup 1.0000x reference)
import functools

import jax
import jax.numpy as jnp
from jax import lax
from jax.experimental import pallas as pl
from jax.experimental.pallas import tpu as pltpu
from jax.experimental.pallas import tpu_sc as plsc

N = 10000
E = 320000
D = 128
C = 40

NC = 2
NS = 16

NP = 10240
ROWS_PER_TILE = NP // NS
K = 80
NBUF = 4
DG = 16
DH = D // NC
EPT = E // NS
CHUNKS = EPT // K


def _zero_buf(buf, nrows, width):
    zv = jnp.zeros((16,), jnp.float32)

    def zrow(i, _):
        for j in range(width // 16):
            buf[i, pl.ds(j * 16, 16)] = zv
        return 0

    lax.fori_loop(0, nrows, zrow, 0, unroll=False)


def _fill_ones(buf, nrows, width):
    ov = jnp.ones((16,), jnp.float32)

    def orow(i, _):
        for j in range(width // 16):
            buf[i, pl.ds(j * 16, 16)] = ov
        return 0

    lax.fori_loop(0, nrows, orow, 0, unroll=False)


def _zero_acc(zsrc, acc, s):
    def zacc(t, _):
        pltpu.sync_copy(zsrc, acc.at[pl.ds(s * ROWS_PER_TILE + t * K, K)])
        return 0

    lax.fori_loop(0, ROWS_PER_TILE // K, zacc, 0, unroll=False)


def _make_spass(with_deg):
    mesh = plsc.VectorSubcoreMesh(core_axis_name="c", subcore_axis_name="s")

    out_type = jax.ShapeDtypeStruct((NP, D), jnp.float32)
    if with_deg:
        out_type = [out_type, jax.ShapeDtypeStruct((NP, D), jnp.float32)]

    scratch = (
        [pltpu.VMEM((K,), jnp.int32) for _ in range(NBUF)]
        + [pltpu.VMEM((K,), jnp.int32) for _ in range(NBUF)]
        + [pltpu.VMEM((K, DH), jnp.float32) for _ in range(NBUF)]
        + [pltpu.VMEM_SHARED((NP, DH), jnp.float32)]
        + [pltpu.VMEM_SHARED((NP, DH), jnp.float32)]
        + [pltpu.SemaphoreType.DMA for _ in range(4 * NBUF)]
        + ([pltpu.VMEM((K, DG), jnp.float32),
            pltpu.VMEM_SHARED((NP, DG), jnp.float32)]
           + [pltpu.SemaphoreType.DMA for _ in range(NBUF)]
           if with_deg else [])
    )

    @functools.partial(
        pl.kernel,
        out_type=out_type,
        mesh=mesh,
        scratch_types=scratch,
        compiler_params=pltpu.CompilerParams(use_tc_tiling_on_sc=False),
    )
    def spass(h_hbm, src_hbm, dst_hbm, *rest):
        if with_deg:
            out_hbm, deg_hbm = rest[0], rest[1]
            sc = rest[2:]
        else:
            out_hbm = rest[0]
            sc = rest[1:]
        srcb = sc[0:NBUF]
        dstb = sc[NBUF:2 * NBUF]
        rows = sc[2 * NBUF:3 * NBUF]
        acc = sc[3 * NBUF]
        hloc = sc[3 * NBUF + 1]
        o = 3 * NBUF + 2
        isems = sc[o:o + NBUF]
        idsems = sc[o + NBUF:o + 2 * NBUF]
        gsems = sc[o + 2 * NBUF:o + 3 * NBUF]
        ssems = sc[o + 3 * NBUF:o + 4 * NBUF]
        if with_deg:
            ones = sc[o + 4 * NBUF]
            dacc = sc[o + 4 * NBUF + 1]
            dsems = sc[o + 4 * NBUF + 2:o + 5 * NBUF + 2]

        c = lax.axis_index("c")
        s = lax.axis_index("s")
        base = s * EPT

        _zero_buf(rows[0], K, DH)
        _zero_acc(rows[0], acc, s)
        if with_deg:
            _zero_buf(ones, K, DG)
            _zero_acc(ones, dacc, s)
            _fill_ones(ones, K, DG)
        rs = pl.ds(s * ROWS_PER_TILE, ROWS_PER_TILE)
        cs = pl.ds(c * DH, DH)
        last = (NS - 1) * ROWS_PER_TILE

        @pl.when(s < NS - 1)
        def _stage_full():
            pltpu.sync_copy(h_hbm.at[rs, cs], hloc.at[rs])

        @pl.when(s == NS - 1)
        def _stage_tail():
            pltpu.sync_copy(h_hbm.at[pl.ds(last, N - last), cs],
                            hloc.at[pl.ds(last, N - last)])

        plsc.subcore_barrier()

        def issue_idx(g, b):
            off = base + g * K
            pltpu.async_copy(src_hbm.at[pl.ds(off, K)], srcb[b], isems[b])
            pltpu.async_copy(dst_hbm.at[pl.ds(off, K)], dstb[b], idsems[b])

        def wait_idx(g, b):
            off = base + g * K
            pltpu.make_async_copy(src_hbm.at[pl.ds(off, K)], srcb[b],
                                  isems[b]).wait()
            pltpu.make_async_copy(dst_hbm.at[pl.ds(off, K)], dstb[b],
                                  idsems[b]).wait()

        def issue_gather(b):
            pltpu.async_copy(hloc.at[srcb[b]], rows[b], gsems[b])

        def wait_gather(b):
            pltpu.make_async_copy(hloc.at[srcb[b]], rows[b], gsems[b]).wait()

        deg_on = jnp.logical_or(
            jnp.logical_and(c == 0, s < NS // 2),
            jnp.logical_and(c == 1, s >= NS // 2)) if with_deg else None

        def issue_scatter(b):
            pltpu.async_copy(rows[b], acc.at[dstb[b]], ssems[b], add=True)
            if with_deg:
                @pl.when(deg_on)
                def _():
                    pltpu.async_copy(ones, dacc.at[dstb[b]], dsems[b],
                                     add=True)

        def wait_scatter(b):
            pltpu.make_async_copy(rows[b], acc.at[dstb[b]], ssems[b]).wait()
            if with_deg:
                @pl.when(deg_on)
                def _():
                    pltpu.make_async_copy(ones, dacc.at[dstb[b]],
                                          dsems[b]).wait()

        issue_idx(0, 0)
        issue_idx(1, 1)
        wait_idx(0, 0)
        issue_gather(0)
        issue_idx(2, 2)
        wait_idx(1, 1)
        issue_gather(1)
        wait_gather(0)
        issue_scatter(0)
        issue_idx(3, 3)
        wait_idx(2, 2)
        issue_gather(2)
        wait_gather(1)
        issue_scatter(1)

        def group(grp, _):
            for b in range(NBUF):
                g = NBUF + grp * NBUF + b
                wait_scatter(b)
                issue_idx(g, b)
                wait_idx(g - 1, (b - 1) % NBUF)
                issue_gather((b - 1) % NBUF)
                wait_gather((b - 2) % NBUF)
                issue_scatter((b - 2) % NBUF)
            return 0

        n_steady = (CHUNKS - 6) // NBUF if CHUNKS % 4 == 2 else (CHUNKS - 4) // NBUF
        lax.fori_loop(0, n_steady, group, 0, unroll=False)

        if CHUNKS % 4 == 2:
            wait_scatter(0)
            issue_idx(CHUNKS - 2, 0)
            wait_idx(CHUNKS - 3, 3)
            issue_gather(3)
            wait_gather(2)
            issue_scatter(2)
            wait_scatter(1)
            issue_idx(CHUNKS - 1, 1)
            wait_idx(CHUNKS - 2, 0)
            issue_gather(0)
            wait_gather(3)
            issue_scatter(3)
            wait_scatter(2)
            wait_idx(CHUNKS - 1, 1)
            issue_gather(1)
            wait_gather(0)
            issue_scatter(0)
            wait_scatter(3)
            wait_gather(1)
            issue_scatter(1)
            wait_scatter(0)
            wait_scatter(1)
        else:
            assert CHUNKS % 4 == 0
            wait_scatter(0)
            wait_idx(CHUNKS - 1, 3)
            issue_gather(3)
            wait_gather(2)
            issue_scatter(2)
            wait_scatter(1)
            wait_gather(3)
            issue_scatter(3)
            wait_scatter(2)
            wait_scatter(3)

        plsc.subcore_barrier()
        pltpu.sync_copy(acc.at[rs], out_hbm.at[rs, cs])
        if with_deg:
            pltpu.sync_copy(dacc.at[rs], deg_hbm.at[rs, pl.ds(c * DG, DG)])

    return spass


_spass_a = _make_spass(True)
_spass = _make_spass(False)


BR = 2000


def _tc_call(body, row_widths, weight_shapes, out_widths):
    grid = (N // BR,)

    def rows_spec(w):
        return pl.BlockSpec((BR, w), lambda i: (i, 0))

    def whole_spec(shape):
        return pl.BlockSpec(shape, lambda i: tuple(0 for _ in shape))

    in_specs = [rows_spec(w) for w in row_widths] + [
        whole_spec(sh) for sh in weight_shapes
    ]
    out_specs = [rows_spec(w) for w in out_widths]
    out_shape = [jax.ShapeDtypeStruct((N, w), jnp.float32) for w in out_widths]
    if len(out_widths) == 1:
        out_specs = out_specs[0]
        out_shape = out_shape[0]
    return pl.pallas_call(
        body, grid=grid, in_specs=in_specs, out_specs=out_specs,
        out_shape=out_shape,
    )


def _dinv(deg):
    d = deg[...]
    return 1.0 / jnp.maximum(d[:, 0:1] + d[:, DG:DG + 1], 1.0)


def _tc1_body(x_ref, s1, deg, w0, w1, b, hp1_out, part_out):
    hp1 = s1[...] * _dinv(deg)
    part = (
        jnp.dot(x_ref[...], w0[...], preferred_element_type=jnp.float32)
        + jnp.dot(hp1, w1[...], preferred_element_type=jnp.float32)
        + b[...]
    )
    hp1_out[...] = hp1
    part_out[...] = part


def _layer_norm(h, g, b):
    m = jnp.mean(h, axis=-1, keepdims=True)
    v = jnp.mean((h - m) ** 2, axis=-1, keepdims=True)
    return (h - m) * lax.rsqrt(v + 1e-5) * g + b


def _tc2_body(part, s2, deg, w2, g, b, h_out):
    hp2 = s2[...] * _dinv(deg)
    h = part[...] + jnp.dot(hp2, w2[...], preferred_element_type=jnp.float32)
    h_out[...] = _layer_norm(h, g[...], b[...])


def _tc4_body(part, s4, deg, w2, g, b, wlin, blin, out_ref):
    hp2 = s4[...] * _dinv(deg)
    h = part[...] + jnp.dot(hp2, w2[...], preferred_element_type=jnp.float32)
    h = _layer_norm(h, g[...], b[...])
    logits = jnp.dot(h, wlin[...], preferred_element_type=jnp.float32) + blin[...]
    m = jnp.max(logits, axis=-1, keepdims=True)
    sh = logits - m
    lse = jnp.log(jnp.sum(jnp.exp(sh), axis=-1, keepdims=True))
    out_ref[...] = sh - lse


def kernel(x, edge_index, W0, b0, W1, b1, ln0_g, ln0_b, ln1_g, ln1_b, Wlin, blin):
    b0r = b0.reshape(1, D)
    b1r = b1.reshape(1, D)

    esrc = edge_index[0]
    edst = edge_index[1]

    s1, deg = _spass_a(x, esrc, edst)

    tc1 = _tc_call(_tc1_body, [D, D, D],
                   [(D, D), (D, D), (1, D)], [D, D])
    hp1, part0 = tc1(x, s1, deg, W0[0], W0[1], b0r)

    s2 = _spass(hp1, esrc, edst)
    tc2 = _tc_call(_tc2_body, [D, D, D],
                   [(D, D), (1, D), (1, D)], [D])
    h0 = tc2(part0, s2, deg, W0[2],
             ln0_g.reshape(1, D), ln0_b.reshape(1, D))

    s3 = _spass(h0, esrc, edst)
    tc3 = _tc_call(_tc1_body, [D, D, D],
                   [(D, D), (D, D), (1, D)], [D, D])
    hp1b, part1 = tc3(h0, s3, deg, W1[0], W1[1], b1r)

    s4 = _spass(hp1b, esrc, edst)
    wlin_pad = jnp.pad(Wlin, ((0, 0), (0, D - C)))
    blin_pad = jnp.concatenate(
        [blin, jnp.full((D - C,), -1e30, jnp.float32)]).reshape(1, D)
    tc4 = _tc_call(_tc4_body, [D, D, D],
                   [(D, D), (1, D), (1, D), (D, D), (1, D)], [D])
    out = tc4(part1, s4, deg, W1[2],
              ln1_g.reshape(1, D), ln1_b.reshape(1, D), wlin_pad, blin_pad)

    return out[:, :C]


# speedup vs baseline: 1.0385x; 1.0385x over previous
import functools

import jax
import jax.numpy as jnp
from jax import lax
from jax.experimental import pallas as pl
from jax.experimental.pallas import tpu as pltpu
from jax.experimental.pallas import tpu_sc as plsc

N = 10000
E = 320000
D = 128
C = 40

NC = 2
NS = 16

NP = 10240
ROWS_PER_TILE = NP // NS
K = 80
NBUF = 5
DG = 16
DH = D // NC
EPT = E // NS
CHUNKS = EPT // K


def _zero_buf(buf, nrows, width):
    zv = jnp.zeros((16,), jnp.float32)

    def zrow(i, _):
        for j in range(width // 16):
            buf[i, pl.ds(j * 16, 16)] = zv
        return 0

    lax.fori_loop(0, nrows, zrow, 0, unroll=False)


def _fill_ones(buf, nrows, width):
    ov = jnp.ones((16,), jnp.float32)

    def orow(i, _):
        for j in range(width // 16):
            buf[i, pl.ds(j * 16, 16)] = ov
        return 0

    lax.fori_loop(0, nrows, orow, 0, unroll=False)


def _zero_acc(zsrc, acc, s):
    def zacc(t, _):
        pltpu.sync_copy(zsrc, acc.at[pl.ds(s * ROWS_PER_TILE + t * K, K)])
        return 0

    lax.fori_loop(0, ROWS_PER_TILE // K, zacc, 0, unroll=False)


def _make_spass(with_deg):
    mesh = plsc.VectorSubcoreMesh(core_axis_name="c", subcore_axis_name="s")

    out_type = jax.ShapeDtypeStruct((NP, D), jnp.float32)
    if with_deg:
        out_type = [out_type, jax.ShapeDtypeStruct((NP, D), jnp.float32)]

    scratch = (
        [pltpu.VMEM((K,), jnp.int32) for _ in range(NBUF)]
        + [pltpu.VMEM((K,), jnp.int32) for _ in range(NBUF)]
        + [pltpu.VMEM((K, DH), jnp.float32) for _ in range(NBUF)]
        + [pltpu.VMEM_SHARED((NP, DH), jnp.float32)]
        + [pltpu.VMEM_SHARED((NP, DH), jnp.float32)]
        + [pltpu.SemaphoreType.DMA for _ in range(4 * NBUF)]
        + ([pltpu.VMEM((K, DG), jnp.float32),
            pltpu.VMEM_SHARED((NP, DG), jnp.float32)]
           + [pltpu.SemaphoreType.DMA for _ in range(NBUF)]
           if with_deg else [])
    )

    @functools.partial(
        pl.kernel,
        out_type=out_type,
        mesh=mesh,
        scratch_types=scratch,
        compiler_params=pltpu.CompilerParams(use_tc_tiling_on_sc=False),
    )
    def spass(h_hbm, edge_hbm, *rest):
        if with_deg:
            out_hbm, deg_hbm = rest[0], rest[1]
            sc = rest[2:]
        else:
            out_hbm = rest[0]
            sc = rest[1:]
        srcb = sc[0:NBUF]
        dstb = sc[NBUF:2 * NBUF]
        rows = sc[2 * NBUF:3 * NBUF]
        acc = sc[3 * NBUF]
        hloc = sc[3 * NBUF + 1]
        o = 3 * NBUF + 2
        isems = sc[o:o + NBUF]
        idsems = sc[o + NBUF:o + 2 * NBUF]
        gsems = sc[o + 2 * NBUF:o + 3 * NBUF]
        ssems = sc[o + 3 * NBUF:o + 4 * NBUF]
        if with_deg:
            ones = sc[o + 4 * NBUF]
            dacc = sc[o + 4 * NBUF + 1]
            dsems = sc[o + 4 * NBUF + 2:o + 5 * NBUF + 2]

        c = lax.axis_index("c")
        s = lax.axis_index("s")
        base = s * EPT

        _zero_buf(rows[0], K, DH)
        _zero_acc(rows[0], acc, s)
        if with_deg:
            _zero_buf(ones, K, DG)
            _zero_acc(ones, dacc, s)
            _fill_ones(ones, K, DG)
        rs = pl.ds(s * ROWS_PER_TILE, ROWS_PER_TILE)
        cs = pl.ds(c * DH, DH)
        last = (NS - 1) * ROWS_PER_TILE

        @pl.when(s < NS - 1)
        def _stage_full():
            pltpu.sync_copy(h_hbm.at[rs, cs], hloc.at[rs])

        @pl.when(s == NS - 1)
        def _stage_tail():
            pltpu.sync_copy(h_hbm.at[pl.ds(last, N - last), cs],
                            hloc.at[pl.ds(last, N - last)])

        plsc.subcore_barrier()

        def issue_idx(g, b):
            off = base + g * K
            pltpu.async_copy(edge_hbm.at[0, pl.ds(off, K)], srcb[b], isems[b])
            pltpu.async_copy(edge_hbm.at[1, pl.ds(off, K)], dstb[b], idsems[b])

        def wait_idx(g, b):
            off = base + g * K
            pltpu.make_async_copy(edge_hbm.at[0, pl.ds(off, K)], srcb[b],
                                  isems[b]).wait()
            pltpu.make_async_copy(edge_hbm.at[1, pl.ds(off, K)], dstb[b],
                                  idsems[b]).wait()

        def issue_gather(b):
            pltpu.async_copy(hloc.at[srcb[b]], rows[b], gsems[b])

        def wait_gather(b):
            pltpu.make_async_copy(hloc.at[srcb[b]], rows[b], gsems[b]).wait()

        deg_on = jnp.logical_or(
            jnp.logical_and(c == 0, s < NS // 2),
            jnp.logical_and(c == 1, s >= NS // 2)) if with_deg else None

        def issue_scatter(b):
            pltpu.async_copy(rows[b], acc.at[dstb[b]], ssems[b], add=True)
            if with_deg:
                @pl.when(deg_on)
                def _():
                    pltpu.async_copy(ones, dacc.at[dstb[b]], dsems[b],
                                     add=True)

        def wait_scatter(b):
            pltpu.make_async_copy(rows[b], acc.at[dstb[b]], ssems[b]).wait()
            if with_deg:
                @pl.when(deg_on)
                def _():
                    pltpu.make_async_copy(ones, dacc.at[dstb[b]],
                                          dsems[b]).wait()

        assert CHUNKS % NBUF == 0

        for g in range(NBUF):
            issue_idx(g, g)
            if g >= 1:
                wait_idx(g - 1, g - 1)
                issue_gather(g - 1)
            if g >= 2:
                wait_gather(g - 2)
                issue_scatter(g - 2)

        def group(grp, _):
            for b in range(NBUF):
                g = NBUF + grp * NBUF + b
                wait_scatter(b)
                issue_idx(g, b)
                wait_idx(g - 1, (b - 1) % NBUF)
                issue_gather((b - 1) % NBUF)
                wait_gather((b - 2) % NBUF)
                issue_scatter((b - 2) % NBUF)
            return 0

        lax.fori_loop(0, CHUNKS // NBUF - 1, group, 0, unroll=False)

        wait_scatter(0)
        wait_idx(CHUNKS - 1, NBUF - 1)
        issue_gather(NBUF - 1)
        wait_gather(NBUF - 2)
        issue_scatter(NBUF - 2)
        wait_scatter(1)
        wait_gather(NBUF - 1)
        issue_scatter(NBUF - 1)
        for j in range(2, NBUF):
            wait_scatter(j)

        plsc.subcore_barrier()
        pltpu.sync_copy(acc.at[rs], out_hbm.at[rs, cs])
        if with_deg:
            pltpu.sync_copy(dacc.at[rs], deg_hbm.at[rs, pl.ds(c * DG, DG)])

    return spass


_spass_a = _make_spass(True)
_spass = _make_spass(False)


BR = 2000


def _tc_call(body, row_widths, weight_shapes, out_widths):
    grid = (N // BR,)

    def rows_spec(w):
        return pl.BlockSpec((BR, w), lambda i: (i, 0))

    def whole_spec(shape):
        return pl.BlockSpec(shape, lambda i: tuple(0 for _ in shape))

    in_specs = [rows_spec(w) for w in row_widths] + [
        whole_spec(sh) for sh in weight_shapes
    ]
    out_specs = [rows_spec(w) for w in out_widths]
    out_shape = [jax.ShapeDtypeStruct((N, w), jnp.float32) for w in out_widths]
    if len(out_widths) == 1:
        out_specs = out_specs[0]
        out_shape = out_shape[0]
    return pl.pallas_call(
        body, grid=grid, in_specs=in_specs, out_specs=out_specs,
        out_shape=out_shape,
    )


def _dinv(deg):
    d = deg[...]
    return 1.0 / jnp.maximum(d[:, 0:1] + d[:, DG:DG + 1], 1.0)


def _tc1_body(x_ref, s1, deg, w0, w1, b, hp1_out, part_out):
    hp1 = s1[...] * _dinv(deg)
    part = (
        jnp.dot(x_ref[...], w0[...], preferred_element_type=jnp.float32)
        + jnp.dot(hp1, w1[...], preferred_element_type=jnp.float32)
        + b[...]
    )
    hp1_out[...] = hp1
    part_out[...] = part


def _layer_norm(h, g, b):
    m = jnp.mean(h, axis=-1, keepdims=True)
    v = jnp.mean((h - m) ** 2, axis=-1, keepdims=True)
    return (h - m) * lax.rsqrt(v + 1e-5) * g + b


def _tc2_body(part, s2, deg, w2, g, b, h_out):
    hp2 = s2[...] * _dinv(deg)
    h = part[...] + jnp.dot(hp2, w2[...], preferred_element_type=jnp.float32)
    h_out[...] = _layer_norm(h, g[...], b[...])


def _tc4_body(part, s4, deg, w2, g, b, wlin, blin, out_ref):
    hp2 = s4[...] * _dinv(deg)
    h = part[...] + jnp.dot(hp2, w2[...], preferred_element_type=jnp.float32)
    h = _layer_norm(h, g[...], b[...])
    logits = jnp.dot(h, wlin[...], preferred_element_type=jnp.float32) + blin[...]
    m = jnp.max(logits, axis=-1, keepdims=True)
    sh = logits - m
    lse = jnp.log(jnp.sum(jnp.exp(sh), axis=-1, keepdims=True))
    out_ref[...] = sh - lse


def kernel(x, edge_index, W0, b0, W1, b1, ln0_g, ln0_b, ln1_g, ln1_b, Wlin, blin):
    b0r = b0.reshape(1, D)
    b1r = b1.reshape(1, D)

    s1, deg = _spass_a(x, edge_index)

    tc1 = _tc_call(_tc1_body, [D, D, D],
                   [(D, D), (D, D), (1, D)], [D, D])
    hp1, part0 = tc1(x, s1, deg, W0[0], W0[1], b0r)

    s2 = _spass(hp1, edge_index)
    tc2 = _tc_call(_tc2_body, [D, D, D],
                   [(D, D), (1, D), (1, D)], [D])
    h0 = tc2(part0, s2, deg, W0[2],
             ln0_g.reshape(1, D), ln0_b.reshape(1, D))

    s3 = _spass(h0, edge_index)
    tc3 = _tc_call(_tc1_body, [D, D, D],
                   [(D, D), (D, D), (1, D)], [D, D])
    hp1b, part1 = tc3(h0, s3, deg, W1[0], W1[1], b1r)

    s4 = _spass(hp1b, edge_index)
    wlin_pad = jnp.pad(Wlin, ((0, 0), (0, D - C)))
    blin_pad = jnp.concatenate(
        [blin, jnp.full((D - C,), -1e30, jnp.float32)]).reshape(1, D)
    tc4 = _tc_call(_tc4_body, [D, D, D],
                   [(D, D), (1, D), (1, D), (D, D), (1, D)], [D])
    out = tc4(part1, s4, deg, W1[2],
              ln1_g.reshape(1, D), ln1_b.reshape(1, D), wlin_pad, blin_pad)

    return out[:, :C]
